# baseline (speedup 1.0000x reference)
import jax
import jax.numpy as jnp
from jax import lax
from jax.experimental import pallas as pl
from jax.experimental.pallas import tpu as pltpu
from jax.experimental.pallas import tpu_sc as plsc

N = 10000
E = 320000
D_IN = 128
D_HID = 16
D_OUT = 40

NROWS = 10240
CHUNK = 2000
CPT32 = 5
EPT = CHUNK * CPT32
RPT = NROWS // 16

_mesh = plsc.VectorSubcoreMesh(core_axis_name="c", subcore_axis_name="s")


def _deg_body(dst_hbm, zero_hbm, deg_hbm, idx_v, acc_v, part_sh, col_v):
    c = lax.axis_index("c")
    s = lax.axis_index("s")
    w = c * 16 + s
    pltpu.sync_copy(dst_hbm.at[pl.ds(w * EPT, EPT)], idx_v)
    pltpu.sync_copy(zero_hbm, acc_v)

    ones16 = jnp.full((16,), 1.0, jnp.float32)

    def cloop(r, carry):
        for k in range(8):
            idx = idx_v[pl.ds(r * 128 + k * 16, 16)]
            plsc.addupdate_scatter(acc_v, [idx], ones16)
        return carry

    lax.fori_loop(0, EPT // 128, cloop, None)

    pltpu.sync_copy(acc_v, part_sh.at[s])
    plsc.subcore_barrier()
    for r in range(16):
        pltpu.sync_copy(part_sh.at[r, pl.ds(s * RPT, RPT)], col_v.at[r])

    def sloop(i, carry):
        tot = jnp.zeros((16,), jnp.float32)
        for r in range(16):
            tot = tot + col_v[r, pl.ds(i * 16, 16)]
        acc_v[pl.ds(i * 16, 16)] = tot
        return carry

    lax.fori_loop(0, RPT // 16, sloop, None)
    pltpu.sync_copy(acc_v.at[pl.ds(0, RPT)], deg_hbm.at[c, pl.ds(s * RPT, RPT)])


_deg_call = pl.kernel(
    _deg_body,
    out_type=jax.ShapeDtypeStruct((2, NROWS), jnp.float32),
    mesh=_mesh,
    compiler_params=pltpu.CompilerParams(needs_layout_passes=False, use_tc_tiling_on_sc=False),
    scratch_types=[
        pltpu.VMEM((EPT,), jnp.int32),
        pltpu.VMEM((NROWS,), jnp.float32),
        pltpu.VMEM_SHARED((16, NROWS), jnp.float32),
        pltpu.VMEM((16, RPT), jnp.float32),
    ],
)


NBUF = 4


def _make_agg(scale):
    def body(h_hbm, deg_hbm, src_hbm, dst_hbm, zero_hbm, out_hbm, dis_hbm,
             sidx, didx, b0, b1, stage, disv, dbuf, ybuf, table_sh, acc_sh,
             g0, g1, s0, s1):
        c = lax.axis_index("c")
        s = lax.axis_index("s")
        w = c * 16 + s
        pltpu.sync_copy(src_hbm.at[pl.ds(w * EPT, EPT)], sidx)
        pltpu.sync_copy(dst_hbm.at[pl.ds(w * EPT, EPT)], didx)

        pltpu.sync_copy(h_hbm.at[pl.ds(s * RPT, RPT)], stage)
        if scale:
            pltpu.sync_copy(deg_hbm.at[0, pl.ds(s * RPT, RPT)],
                            dbuf.at[0])
            pltpu.sync_copy(deg_hbm.at[1, pl.ds(s * RPT, RPT)],
                            dbuf.at[1])

            def dloop(i, carry):
                deg = jnp.full((16,), 1.0, jnp.float32)
                deg = deg + dbuf[0, pl.ds(i * 16, 16)]
                deg = deg + dbuf[1, pl.ds(i * 16, 16)]
                bi = plsc.bitcast(deg, jnp.int32)
                bi = 0x5F3759DF - lax.shift_right_arithmetic(bi, 1)
                y = plsc.bitcast(bi, jnp.float32)
                for _n in range(3):
                    y = y * (1.5 - 0.5 * deg * y * y)
                ybuf[...] = y
                for u in range(16):
                    d = plsc.load_gather(ybuf, [jnp.full((16,), u, jnp.int32)])
                    disv[2 * i + (u // 8), pl.ds((u % 8) * 16, 16)] = d
                    r = i * 16 + u
                    stage[r] = stage[r] * d
                return carry

            lax.fori_loop(0, RPT // 16, dloop, None)

            @pl.when(c == 0)
            def _():
                pltpu.sync_copy(disv, dis_hbm.at[pl.ds(s * (RPT // 8),
                                                       RPT // 8)])
        pltpu.sync_copy(stage, table_sh.at[pl.ds(s * RPT, RPT)])

        @pl.when(c != 0)
        def _():
            pltpu.sync_copy(zero_hbm, stage)

        pltpu.sync_copy(stage, acc_sh.at[pl.ds(s * RPT, RPT)])
        plsc.subcore_barrier()

        bufs = (b0, b1)
        gsems = (g0, g1)
        ssems = (s0, s1)

        def idx_g(m):
            return sidx.at[pl.ds(m * CHUNK, CHUNK)]

        def idx_s(m):
            return didx.at[pl.ds(m * CHUNK, CHUNK)]

        def fire_g(m, u):
            pltpu.async_copy(table_sh.at[idx_g(m)], bufs[u], gsems[u])

        def wait_g(m, u):
            pltpu.make_async_copy(table_sh.at[idx_g(m)], bufs[u],
                                  gsems[u]).wait()

        def fire_s(m, u):
            pltpu.async_copy(bufs[u], acc_sh.at[idx_s(m)], ssems[u],
                             add=True)

        def wait_s(m, u):
            pltpu.make_async_copy(bufs[u], acc_sh.at[idx_s(m)],
                                  ssems[u]).wait()

        fire_g(0, 0)
        fire_g(1, 1)
        for m in range(CPT32):
            u = m % 2
            wait_g(m, u)
            fire_s(m, u)
            if m + 2 < CPT32:
                wait_s(m, u)
                fire_g(m + 2, u)
        for m in range(CPT32 - 2, CPT32):
            wait_s(m, m % 2)

        plsc.subcore_barrier()
        pltpu.sync_copy(acc_sh.at[pl.ds(s * RPT, RPT)], stage)
        pltpu.sync_copy(stage, out_hbm.at[c, pl.ds(s * RPT, RPT)])

    outs = (jax.ShapeDtypeStruct((2, NROWS, D_HID), jnp.float32),
            jax.ShapeDtypeStruct((NROWS // 8, 128), jnp.float32))
    if not scale:
        outs = (jax.ShapeDtypeStruct((2, NROWS, D_HID), jnp.float32),
                jax.ShapeDtypeStruct((8, 128), jnp.float32))
    return pl.kernel(
        body,
        out_type=outs,
        mesh=_mesh,
        compiler_params=pltpu.CompilerParams(needs_layout_passes=False,
                                             use_tc_tiling_on_sc=False),
        scratch_types=[
            pltpu.VMEM((EPT,), jnp.int32),
            pltpu.VMEM((EPT,), jnp.int32),
            pltpu.VMEM((CHUNK, D_HID), jnp.float32),
            pltpu.VMEM((CHUNK, D_HID), jnp.float32),
            pltpu.VMEM((RPT, D_HID), jnp.float32),
            pltpu.VMEM((RPT // 8, 128), jnp.float32),
            pltpu.VMEM((2, RPT), jnp.float32),
            pltpu.VMEM((16,), jnp.float32),
            pltpu.VMEM_SHARED((NROWS, D_HID), jnp.float32),
            pltpu.VMEM_SHARED((NROWS, D_HID), jnp.float32),
            pltpu.SemaphoreType.DMA,
            pltpu.SemaphoreType.DMA,
            pltpu.SemaphoreType.DMA,
            pltpu.SemaphoreType.DMA,
        ],
    )


_agg_scaled_call = _make_agg(True)
_agg_plain_call = _make_agg(False)


def _k2_body(x_ref, w_ref, o_ref):
    o_ref[pl.ds(0, N), :] = jnp.dot(x_ref[...], w_ref[...],
                                    preferred_element_type=jnp.float32)
    o_ref[pl.ds(N, NROWS - N), :] = jnp.zeros((NROWS - N, D_HID), jnp.float32)


def _k4_body(parts_ref, dis_ref, b_ref, o_ref):
    agg = parts_ref[0] + parts_ref[1]
    z = jnp.maximum(agg * dis_ref[...] + b_ref[...], 0.0)
    o_ref[...] = z * dis_ref[...]


def _k6_body(parts_ref, dis_ref, w_ref, b_ref, o_ref):
    u = (parts_ref[0] + parts_ref[1]) * dis_ref[...]
    o = jnp.dot(u, w_ref[...], preferred_element_type=jnp.float32) + b_ref[...]
    o3 = o.reshape(o.shape[0], 8, D_OUT)
    m = jnp.max(o3, axis=2, keepdims=True)
    e = o3 - m
    lse = jnp.log(jnp.sum(jnp.exp(e), axis=2, keepdims=True))
    o_ref[...] = (e - lse).reshape(o.shape[0], 8 * D_OUT)


NRV = NROWS // 8

_k2_call = pl.pallas_call(
    _k2_body, out_shape=jax.ShapeDtypeStruct((NROWS, D_HID), jnp.float32))

_K4R = 320
_k4_call = pl.pallas_call(
    _k4_body,
    grid=(NRV // _K4R,),
    in_specs=[
        pl.BlockSpec((2, _K4R, 128), lambda i: (0, i, 0)),
        pl.BlockSpec((_K4R, 128), lambda i: (i, 0)),
        pl.BlockSpec((1, 128), lambda i: (0, 0)),
    ],
    out_specs=pl.BlockSpec((_K4R, 128), lambda i: (i, 0)),
    out_shape=jax.ShapeDtypeStruct((NRV, 128), jnp.float32))

_K6R = 320
_k6_call = pl.pallas_call(
    _k6_body,
    grid=(NRV // _K6R,),
    in_specs=[
        pl.BlockSpec((2, _K6R, 128), lambda i: (0, i, 0)),
        pl.BlockSpec((_K6R, 128), lambda i: (i, 0)),
        pl.BlockSpec((128, 8 * D_OUT), lambda i: (0, 0)),
        pl.BlockSpec((1, 8 * D_OUT), lambda i: (0, 0)),
    ],
    out_specs=pl.BlockSpec((_K6R, 8 * D_OUT), lambda i: (i, 0)),
    out_shape=jax.ShapeDtypeStruct((NRV, 8 * D_OUT), jnp.float32))


@jax.jit
def kernel(x, edge_index, W1, b1, W2, b2):
    h1 = _k2_call(x, W1)
    zdeg = jnp.zeros((NROWS,), jnp.float32)
    srcv = edge_index[0]
    dstv = edge_index[1]
    deg2 = _deg_call(dstv, zdeg)
    zrows = jnp.zeros((RPT, D_HID), jnp.float32)
    parts1, dis16 = _agg_scaled_call(h1, deg2, srcv, dstv, zrows)
    b1t = jnp.tile(b1, (8,)).reshape(1, 128)
    zp = _k4_call(parts1.reshape(2, NRV, 128), dis16, b1t)
    parts2, _unused = _agg_plain_call(zp.reshape(NROWS, D_HID), deg2,
                                      srcv, dstv, zrows)
    w2rep = jnp.kron(jnp.eye(8, dtype=jnp.float32), W2)
    b2t = jnp.tile(b2, (8,)).reshape(1, 8 * D_OUT)
    out320 = _k6_call(parts2.reshape(2, NRV, 128), dis16, w2rep, b2t)
    return out320[:N // 8].reshape(N, D_OUT)
